# baseline (speedup 1.0000x reference)
import functools

import jax
import jax.numpy as jnp
from jax import lax
from jax.experimental import pallas as pl
from jax.experimental.pallas import tpu as pltpu
from jax.experimental.pallas import tpu_sc as plsc

_N = 131072
_B = 4096
_K = 100


_LC = 32768


def _peratom_body(logits_ref, h0_ref, pred_ref, targ_ref, s_ref):
    x = logits_ref[...]
    m = jnp.max(x, axis=0, keepdims=True)
    se = jnp.sum(jnp.exp(x - m), axis=0, keepdims=True)
    logz = jnp.log(se) + m
    rows = lax.broadcasted_iota(jnp.int32, x.shape, 0)
    picked = jnp.sum(jnp.where(rows == h0_ref[...], x, 0.0), axis=0,
                     keepdims=True)
    d = targ_ref[...] - pred_ref[...]
    sq = jnp.sum(d * d, axis=0, keepdims=True)
    s_ref[...] = ((logz - picked) + sq)[0]


def _stage1(logits_t, h0_row, pred_t, targ_t, off, n):
    nb = off // _LC
    return pl.pallas_call(
        _peratom_body,
        grid=(n // _LC,),
        in_specs=[
            pl.BlockSpec((_K, _LC), lambda i: (0, i + nb)),
            pl.BlockSpec((_LC,), lambda i: (i + nb,)),
            pl.BlockSpec((3, _LC), lambda i: (0, i + nb)),
            pl.BlockSpec((3, _LC), lambda i: (0, i + nb)),
        ],
        out_specs=pl.BlockSpec((_LC,), lambda i: (i,)),
        out_shape=jax.ShapeDtypeStruct((n,), jnp.float32),
    )(logits_t, h0_row, pred_t, targ_t)


_NC = 2
_NS = 16
_NW = _NC * _NS
_ROWS = _N // 128
_RPT = _ROWS // _NW
_SEG_T = _B // _NS


_WAVE = 8


def _make_sc_counts_body(apt):
    def _sc_counts(idx_hbm, out_hbm, idx_flat, ones_v, zeros_v, shared, sem):
        c = lax.axis_index("c")
        t = lax.axis_index("s")
        wid = c * _NS + t

        pltpu.sync_copy(idx_hbm.at[pl.ds(wid * apt, apt)], idx_flat)

        for i in range(apt // 16):
            ones_v[pl.ds(i * 16, 16)] = jnp.ones((16,), jnp.float32)
        for i in range(_SEG_T // 16):
            zeros_v[pl.ds(i * 16, 16)] = jnp.zeros((16,), jnp.float32)

        pltpu.sync_copy(zeros_v, shared.at[pl.ds(t * _SEG_T, _SEG_T)])
        plsc.subcore_barrier()

        pltpu.async_copy(ones_v, shared.at[idx_flat], sem, add=True).wait()
        plsc.subcore_barrier()

        pltpu.sync_copy(shared.at[pl.ds(t * _SEG_T, _SEG_T)],
                        out_hbm.at[c, pl.ds(t * _SEG_T, _SEG_T)])

    return _sc_counts


def _segcounts(idx_flat):
    apt = idx_flat.shape[0] // _NW
    mesh = plsc.VectorSubcoreMesh(core_axis_name="c", subcore_axis_name="s")
    f = pl.kernel(
        _make_sc_counts_body(apt),
        mesh=mesh,
        out_type=jax.ShapeDtypeStruct((_NC, _B), jnp.float32),
        scratch_types=[
            pltpu.VMEM((apt,), jnp.int32),
            pltpu.VMEM((apt,), jnp.float32),
            pltpu.VMEM((_SEG_T,), jnp.float32),
            pltpu.VMEM_SHARED((_B,), jnp.float32),
            pltpu.SemaphoreType.DMA,
        ],
    )
    return f(idx_flat)


def _make_sc_body(apt, off):
    def _sc_body(idx_hbm, val_hbm, out_hbm, idx_flat, val_flat, zeros_v,
                 shared, sem):
        c = lax.axis_index("c")
        t = lax.axis_index("s")
        wid = c * _NS + t

        pltpu.sync_copy(idx_hbm.at[pl.ds(off + wid * apt, apt)], idx_flat)
        pltpu.sync_copy(val_hbm.at[pl.ds(wid * apt, apt)], val_flat)

        for i in range(_SEG_T // 16):
            zeros_v[pl.ds(i * 16, 16)] = jnp.zeros((16,), jnp.float32)

        pltpu.sync_copy(zeros_v, shared.at[pl.ds(t * _SEG_T, _SEG_T)])
        plsc.subcore_barrier()

        pltpu.async_copy(val_flat, shared.at[idx_flat], sem, add=True).wait()
        plsc.subcore_barrier()

        pltpu.sync_copy(shared.at[pl.ds(t * _SEG_T, _SEG_T)],
                        out_hbm.at[c, pl.ds(t * _SEG_T, _SEG_T)])

    return _sc_body


def _segscat(idx_flat, val_flat, off=0):
    apt = val_flat.shape[0] // _NW
    mesh = plsc.VectorSubcoreMesh(core_axis_name="c", subcore_axis_name="s")
    f = pl.kernel(
        _make_sc_body(apt, off),
        mesh=mesh,
        out_type=jax.ShapeDtypeStruct((_NC, _B), jnp.float32),
        scratch_types=[
            pltpu.VMEM((apt,), jnp.int32),
            pltpu.VMEM((apt,), jnp.float32),
            pltpu.VMEM((_SEG_T,), jnp.float32),
            pltpu.VMEM_SHARED((_B,), jnp.float32),
            pltpu.SemaphoreType.DMA,
        ],
    )
    return f(idx_flat, val_flat)


def _combine_body(sums_ref, cnts_ref, svp_ref, svt_ref,
                  latp_ref, latt_ref, out_ref):
    ssum = sums_ref[0:1, :] + sums_ref[1:2, :]
    cnt = jnp.maximum(cnts_ref[0:1, :] + cnts_ref[1:2, :], 1.0)
    exh = jnp.sum(ssum / cnt) * (1.0 / _B)

    dsv = svp_ref[...] - svt_ref[...]
    err_sv = jnp.sum(dsv * dsv) * (1.0 / (_B * 6))

    acc = jnp.float32(0.0)
    for g in range(3):
        p2 = (latp_ref[3 * g:3 * g + 1, :] ** 2
              + latp_ref[3 * g + 1:3 * g + 2, :] ** 2
              + latp_ref[3 * g + 2:3 * g + 3, :] ** 2)
        t2 = (latt_ref[3 * g:3 * g + 1, :] ** 2
              + latt_ref[3 * g + 1:3 * g + 2, :] ** 2
              + latt_ref[3 * g + 2:3 * g + 3, :] ** 2)
        dl = jnp.sqrt(p2 + 1e-12) - jnp.sqrt(t2 + 1e-12)
        acc = acc + jnp.sum(dl * dl)
    err_len = acc * (1.0 / (_B * 3))

    out_ref[0, 0] = exh + err_sv + err_len


def _combine(sums, cnts, svp, svt, latp, latt):
    return pl.pallas_call(
        _combine_body,
        out_specs=pl.BlockSpec(memory_space=pltpu.SMEM),
        out_shape=jax.ShapeDtypeStruct((1, 1), jnp.float32),
    )(sums, cnts, svp, svt, latp, latt)


def kernel(pred_frac_eps_x, target_frac_eps_x, predicted_h0_logits,
           pred_symmetric_vector_noise, symmetric_vector_noise,
           pred_lattice, lattice, batch_idx, h0):
    h0_row = h0.astype(jnp.int32)
    idx_rows = batch_idx.astype(jnp.int32)

    cnts = _segscat(idx_rows, jnp.ones((_N,), jnp.float32), 0)
    s = _stage1(predicted_h0_logits.T, h0_row,
                pred_frac_eps_x.T, target_frac_eps_x.T, 0, _N)
    sums = _segscat(idx_rows, s, 0)

    svp = pred_symmetric_vector_noise.T
    svt = symmetric_vector_noise.T
    latp = pred_lattice.reshape(_B, 9).T
    latt = lattice.reshape(_B, 9).T

    out = _combine(sums, cnts, svp, svt, latp, latt)
    return out[0, 0]


# speedup vs baseline: 1.0093x; 1.0093x over previous
import jax
import jax.numpy as jnp
from jax import lax
from jax.experimental import pallas as pl
from jax.experimental.pallas import tpu as pltpu
from jax.experimental.pallas import tpu_sc as plsc

_N = 131072
_B = 4096
_K = 100


_LC = 16384


def _peratom_body(logits_ref, h0_ref, pred_ref, targ_ref, s_ref):
    x = logits_ref[...]
    m = jnp.max(x, axis=0, keepdims=True)
    se = jnp.sum(jnp.exp(x - m), axis=0, keepdims=True)
    logz = jnp.log(se) + m
    rows = lax.broadcasted_iota(jnp.int32, x.shape, 0)
    picked = jnp.sum(jnp.where(rows == h0_ref[...], x, 0.0), axis=0,
                     keepdims=True)
    d = targ_ref[...] - pred_ref[...]
    sq = jnp.sum(d * d, axis=0, keepdims=True)
    s_ref[...] = ((logz - picked) + sq)[0]


def _stage1(logits_t, h0_row, pred_t, targ_t, off, n):
    nb = off // _LC
    return pl.pallas_call(
        _peratom_body,
        grid=(n // _LC,),
        in_specs=[
            pl.BlockSpec((_K, _LC), lambda i: (0, i + nb)),
            pl.BlockSpec((_LC,), lambda i: (i + nb,)),
            pl.BlockSpec((3, _LC), lambda i: (0, i + nb)),
            pl.BlockSpec((3, _LC), lambda i: (0, i + nb)),
        ],
        out_specs=pl.BlockSpec((_LC,), lambda i: (i,)),
        out_shape=jax.ShapeDtypeStruct((n,), jnp.float32),
    )(logits_t, h0_row, pred_t, targ_t)


_NC = 2
_NS = 16
_NW = _NC * _NS
_SEG_T = _B // _NS


def _make_sc_body(apt, off):
    def _sc_body(idx_hbm, val_hbm, out_hbm, idx_flat, val_flat, zeros_v,
                 shared, sem):
        c = lax.axis_index("c")
        t = lax.axis_index("s")
        wid = c * _NS + t

        pltpu.sync_copy(idx_hbm.at[pl.ds(off + wid * apt, apt)], idx_flat)
        pltpu.sync_copy(val_hbm.at[pl.ds(wid * apt, apt)], val_flat)

        for i in range(_SEG_T // 16):
            zeros_v[pl.ds(i * 16, 16)] = jnp.zeros((16,), jnp.float32)

        pltpu.sync_copy(zeros_v, shared.at[pl.ds(t * _SEG_T, _SEG_T)])
        plsc.subcore_barrier()

        pltpu.async_copy(val_flat, shared.at[idx_flat], sem, add=True).wait()
        plsc.subcore_barrier()

        pltpu.sync_copy(shared.at[pl.ds(t * _SEG_T, _SEG_T)],
                        out_hbm.at[c, pl.ds(t * _SEG_T, _SEG_T)])

    return _sc_body


def _segscat(idx_flat, val_flat, off=0):
    apt = val_flat.shape[0] // _NW
    mesh = plsc.VectorSubcoreMesh(core_axis_name="c", subcore_axis_name="s")
    f = pl.kernel(
        _make_sc_body(apt, off),
        mesh=mesh,
        out_type=jax.ShapeDtypeStruct((_NC, _B), jnp.float32),
        scratch_types=[
            pltpu.VMEM((apt,), jnp.int32),
            pltpu.VMEM((apt,), jnp.float32),
            pltpu.VMEM((_SEG_T,), jnp.float32),
            pltpu.VMEM_SHARED((_B,), jnp.float32),
            pltpu.SemaphoreType.DMA,
        ],
    )
    return f(idx_flat, val_flat)


def _combine_body(sums_ref, cnts_ref, svp_ref, svt_ref,
                  latp_ref, latt_ref, out_ref):
    ssum = sums_ref[0:1, :] + sums_ref[1:2, :]
    cnt = jnp.maximum(cnts_ref[0:1, :] + cnts_ref[1:2, :], 1.0)
    exh = jnp.sum(ssum / cnt) * (1.0 / _B)

    dsv = svp_ref[...] - svt_ref[...]
    err_sv = jnp.sum(dsv * dsv) * (1.0 / (_B * 6))

    acc = jnp.float32(0.0)
    for g in range(3):
        p2 = (latp_ref[3 * g:3 * g + 1, :] ** 2
              + latp_ref[3 * g + 1:3 * g + 2, :] ** 2
              + latp_ref[3 * g + 2:3 * g + 3, :] ** 2)
        t2 = (latt_ref[3 * g:3 * g + 1, :] ** 2
              + latt_ref[3 * g + 1:3 * g + 2, :] ** 2
              + latt_ref[3 * g + 2:3 * g + 3, :] ** 2)
        dl = jnp.sqrt(p2 + 1e-12) - jnp.sqrt(t2 + 1e-12)
        acc = acc + jnp.sum(dl * dl)
    err_len = acc * (1.0 / (_B * 3))

    out_ref[0, 0] = exh + err_sv + err_len


def _combine(sums, cnts, svp, svt, latp, latt):
    return pl.pallas_call(
        _combine_body,
        out_specs=pl.BlockSpec(memory_space=pltpu.SMEM),
        out_shape=jax.ShapeDtypeStruct((1, 1), jnp.float32),
    )(sums, cnts, svp, svt, latp, latt)


def kernel(pred_frac_eps_x, target_frac_eps_x, predicted_h0_logits,
           pred_symmetric_vector_noise, symmetric_vector_noise,
           pred_lattice, lattice, batch_idx, h0):
    h0_row = h0.astype(jnp.int32)
    idx_rows = batch_idx.astype(jnp.int32)

    cnts = _segscat(idx_rows, jnp.ones((_N,), jnp.float32), 0)
    s = _stage1(predicted_h0_logits.T, h0_row,
                pred_frac_eps_x.T, target_frac_eps_x.T, 0, _N)
    sums = _segscat(idx_rows, s, 0)

    svp = pred_symmetric_vector_noise.T
    svt = symmetric_vector_noise.T
    latp = pred_lattice.reshape(_B, 9).T
    latt = lattice.reshape(_B, 9).T

    out = _combine(sums, cnts, svp, svt, latp, latt)
    return out[0, 0]
